# baseline (speedup 1.0000x reference)
import functools

import jax
import jax.numpy as jnp
import numpy as np
from jax import lax
from jax.experimental import pallas as pl
from jax.experimental.pallas import tpu as pltpu
from jax.experimental.pallas import tpu_sc as plsc

K = 16
T = 8
TB = 32


def _box_kernel(xprev_ref, xcur_ref, xnext_ref, win_ref, *, nt):
    tc = pl.program_id(1)
    xe = jnp.concatenate(
        [xprev_ref[0], xcur_ref[0], xnext_ref[0]], axis=0
    )
    tglob = (tc * TB - TB) + jax.lax.broadcasted_iota(jnp.int32, (3 * TB, 1, 1), 0)
    valid = (tglob >= 0) & (tglob < nt * TB)
    xe = jnp.where(valid, xe, 0.0)
    s2 = xe[:-1] + xe[1:]
    s4 = s2[:-2] + s2[2:]
    s8 = s4[:-4] + s4[4:]
    s16 = s8[:-8] + s8[8:]
    win_ref[...] = (s16[TB - T : 2 * TB - T] + xe[TB + T : 2 * TB + T])[None]


def _box_filter(x_in):
    B, Tt, F, C = x_in.shape
    nt = Tt // TB
    xspec = lambda fn: pl.BlockSpec((1, TB, F, C), fn)
    return pl.pallas_call(
        functools.partial(_box_kernel, nt=nt),
        grid=(B, nt),
        in_specs=[
            xspec(lambda b, t: (b, jnp.maximum(t - 1, 0), 0, 0)),
            xspec(lambda b, t: (b, t, 0, 0)),
            xspec(lambda b, t: (b, jnp.minimum(t + 1, nt - 1), 0, 0)),
        ],
        out_specs=pl.BlockSpec((1, TB, F, C), lambda b, t: (b, t, 0, 0)),
        out_shape=jax.ShapeDtypeStruct((B, Tt, F, C), x_in.dtype),
    )(x_in, x_in, x_in)


def _harmonic_mix_sc(win):
    B, Tt, F, C = win.shape
    NLANES = 16
    NCHUNK = C // NLANES
    info = plsc.get_sparse_core_info()
    nworkers = info.num_cores * info.num_subcores
    nslabs = B * Tt
    slabs_per_worker = nslabs // nworkers
    mesh = plsc.VectorSubcoreMesh(core_axis_name="c", subcore_axis_name="s")

    @functools.partial(
        pl.kernel,
        mesh=mesh,
        out_type=jax.ShapeDtypeStruct((B, Tt, F, C), jnp.float32),
        scratch_types=[
            pltpu.VMEM((F, C), jnp.float32),
            pltpu.VMEM((F, C), jnp.float32),
        ],
    )
    def mix(win_hbm, out_hbm, wslab, oslab):
        wid = lax.axis_index("s") * info.num_cores + lax.axis_index("c")

        def slab_body(i, carry):
            s = wid * slabs_per_worker + i
            b = s // Tt
            t = s % Tt
            pltpu.sync_copy(win_hbm.at[b, t], wslab)

            def o_body(o, carry2):
                kmax = jnp.where(
                    o == 0, K, jnp.minimum(K, (F - 1) // jnp.maximum(o, 1))
                )
                cclip = (K - kmax).astype(jnp.float32)
                accs = tuple(
                    wslab[F - 1, pl.ds(c * NLANES, NLANES)] * cclip
                    for c in range(NCHUNK)
                )

                def k_body(k, accs_in):
                    f = k * o
                    return tuple(
                        a + wslab[f, pl.ds(c * NLANES, NLANES)]
                        for c, a in enumerate(accs_in)
                    )

                accs = lax.fori_loop(1, kmax + 1, k_body, accs)
                for c in range(NCHUNK):
                    oslab[o, pl.ds(c * NLANES, NLANES)] = accs[c]
                return carry2

            lax.fori_loop(0, F, o_body, 0)
            pltpu.sync_copy(oslab, out_hbm.at[b, t])
            return carry

        lax.fori_loop(0, slabs_per_worker, slab_body, 0)

    return mix(win)


def kernel(x_in):
    win = _box_filter(x_in)
    return _harmonic_mix_sc(win)


# speedup vs baseline: 1.6111x; 1.6111x over previous
import functools

import jax
import jax.numpy as jnp
import numpy as np
from jax import lax
from jax.experimental import pallas as pl
from jax.experimental.pallas import tpu as pltpu
from jax.experimental.pallas import tpu_sc as plsc

K = 16
T = 8
TB = 32


def _box_kernel(xprev_ref, xcur_ref, xnext_ref, win_ref, *, nt):
    tc = pl.program_id(1)
    xe = jnp.concatenate(
        [xprev_ref[0], xcur_ref[0], xnext_ref[0]], axis=0
    )
    tglob = (tc * TB - TB) + jax.lax.broadcasted_iota(jnp.int32, (3 * TB, 1, 1), 0)
    valid = (tglob >= 0) & (tglob < nt * TB)
    xe = jnp.where(valid, xe, 0.0)
    s2 = xe[:-1] + xe[1:]
    s4 = s2[:-2] + s2[2:]
    s8 = s4[:-4] + s4[4:]
    s16 = s8[:-8] + s8[8:]
    win_ref[...] = (s16[TB - T : 2 * TB - T] + xe[TB + T : 2 * TB + T])[None]


def _box_filter(x_in):
    B, Tt, F, C = x_in.shape
    nt = Tt // TB
    xspec = lambda fn: pl.BlockSpec((1, TB, F, C), fn)
    return pl.pallas_call(
        functools.partial(_box_kernel, nt=nt),
        grid=(B, nt),
        in_specs=[
            xspec(lambda b, t: (b, jnp.maximum(t - 1, 0), 0, 0)),
            xspec(lambda b, t: (b, t, 0, 0)),
            xspec(lambda b, t: (b, jnp.minimum(t + 1, nt - 1), 0, 0)),
        ],
        out_specs=pl.BlockSpec((1, TB, F, C), lambda b, t: (b, t, 0, 0)),
        out_shape=jax.ShapeDtypeStruct((B, Tt, F, C), x_in.dtype),
    )(x_in, x_in, x_in)


def _kmax_groups(F):
    groups = []
    for o in range(F):
        km = K if o == 0 else min(K, (F - 1) // o)
        if groups and groups[-1][2] == km:
            groups[-1] = (groups[-1][0], o + 1, km)
        else:
            groups.append((o, o + 1, km))
    return groups


def _harmonic_mix_sc(win):
    B, Tt, F, C = win.shape
    NLANES = 16
    NCHUNK = C // NLANES
    info = plsc.get_sparse_core_info()
    nworkers = info.num_cores * info.num_subcores
    nslabs = B * Tt
    spw = nslabs // nworkers
    groups = _kmax_groups(F)
    mesh = plsc.VectorSubcoreMesh(core_axis_name="c", subcore_axis_name="s")

    @functools.partial(
        pl.kernel,
        mesh=mesh,
        out_type=jax.ShapeDtypeStruct((B, Tt, F, C), jnp.float32),
        scratch_types=[
            pltpu.VMEM((2, F, C), jnp.float32),
            pltpu.VMEM((2, F, C), jnp.float32),
            pltpu.SemaphoreType.DMA((2,)),
            pltpu.SemaphoreType.DMA((2,)),
        ],
    )
    def mix(win_hbm, out_hbm, wslab, oslab, sem_in, sem_out):
        wid = lax.axis_index("s") * info.num_cores + lax.axis_index("c")
        base = wid * spw

        def in_copy(i):
            s = base + i
            p = i % 2
            return pltpu.make_async_copy(
                win_hbm.at[s // Tt, s % Tt], wslab.at[p], sem_in.at[p]
            )

        def out_copy(i):
            s = base + i
            p = i % 2
            return pltpu.make_async_copy(
                oslab.at[p], out_hbm.at[s // Tt, s % Tt], sem_out.at[p]
            )

        in_copy(0).start()

        def slab_body(i, carry):
            @pl.when(i + 1 < spw)
            def _():
                in_copy(i + 1).start()

            in_copy(i).wait()

            @pl.when(i >= 2)
            def _():
                out_copy(i - 2).wait()

            p = i % 2
            w127 = [
                wslab[p, F - 1, pl.ds(c * NLANES, NLANES)] for c in range(NCHUNK)
            ]
            for (lo, hi, km) in groups:
                cclip = float(K - km)

                def o_body(o, carry2, km=km, cclip=cclip):
                    accs = [w * cclip for w in w127] if km < K else [
                        wslab[p, o, pl.ds(c * NLANES, NLANES)]
                        for c in range(NCHUNK)
                    ]
                    for k in range(1 if km == K else 1, km + 1):
                        if km == K and k == 1:
                            continue
                        accs = [
                            a + wslab[p, k * o, pl.ds(c * NLANES, NLANES)]
                            for c, a in enumerate(accs)
                        ]
                    for c in range(NCHUNK):
                        oslab[p, o, pl.ds(c * NLANES, NLANES)] = accs[c]
                    return carry2

                lax.fori_loop(lo, hi, o_body, 0)
            out_copy(i).start()
            return carry

        lax.fori_loop(0, spw, slab_body, 0)
        out_copy(spw - 2).wait()
        out_copy(spw - 1).wait()

    return mix(win)


def kernel(x_in):
    win = _box_filter(x_in)
    return _harmonic_mix_sc(win)


# speedup vs baseline: 2.0237x; 1.2561x over previous
import functools

import jax
import jax.numpy as jnp
import numpy as np
from jax import lax
from jax.experimental import pallas as pl
from jax.experimental.pallas import tpu as pltpu
from jax.experimental.pallas import tpu_sc as plsc

K = 16
T = 8
TB = 32


FB = 32


def _box_kernel(x_ref, win_ref):
    x = x_ref[0]
    Tt = x.shape[0]
    z = jnp.zeros((T,) + x.shape[1:], x.dtype)
    xp = jnp.concatenate([z, x, z], axis=0)
    s2 = xp[:-1] + xp[1:]
    s4 = s2[:-2] + s2[2:]
    s8 = s4[:-4] + s4[4:]
    s16 = s8[:-8] + s8[8:]
    win_ref[...] = (s16[:Tt] + xp[2 * T :])[None]


def _box_filter(x_in):
    B, Tt, F, C = x_in.shape
    return pl.pallas_call(
        _box_kernel,
        grid=(B, F // FB),
        in_specs=[pl.BlockSpec((1, Tt, FB, C), lambda b, f: (b, 0, f, 0))],
        out_specs=pl.BlockSpec((1, Tt, FB, C), lambda b, f: (b, 0, f, 0)),
        out_shape=jax.ShapeDtypeStruct((B, Tt, F, C), x_in.dtype),
    )(x_in)


def _kmax_groups(F):
    groups = []
    for o in range(F):
        km = K if o == 0 else min(K, (F - 1) // o)
        if groups and groups[-1][2] == km:
            groups[-1] = (groups[-1][0], o + 1, km)
        else:
            groups.append((o, o + 1, km))
    return groups


def _harmonic_mix_sc(win):
    B, Tt, F, C = win.shape
    NLANES = 16
    NCHUNK = C // NLANES
    info = plsc.get_sparse_core_info()
    nworkers = info.num_cores * info.num_subcores
    nslabs = B * Tt
    spw = nslabs // nworkers
    groups = _kmax_groups(F)
    mesh = plsc.VectorSubcoreMesh(core_axis_name="c", subcore_axis_name="s")

    @functools.partial(
        pl.kernel,
        mesh=mesh,
        out_type=jax.ShapeDtypeStruct((B, Tt, F, C), jnp.float32),
        scratch_types=[
            pltpu.VMEM((2, F, C), jnp.float32),
            pltpu.VMEM((2, F, C), jnp.float32),
            pltpu.SemaphoreType.DMA((2,)),
            pltpu.SemaphoreType.DMA((2,)),
        ],
    )
    def mix(win_hbm, out_hbm, wslab, oslab, sem_in, sem_out):
        wid = lax.axis_index("s") * info.num_cores + lax.axis_index("c")
        base = wid * spw

        def in_copy(i):
            s = base + i
            p = i % 2
            return pltpu.make_async_copy(
                win_hbm.at[s // Tt, s % Tt], wslab.at[p], sem_in.at[p]
            )

        def out_copy(i):
            s = base + i
            p = i % 2
            return pltpu.make_async_copy(
                oslab.at[p], out_hbm.at[s // Tt, s % Tt], sem_out.at[p]
            )

        in_copy(0).start()

        def slab_body(i, carry):
            @pl.when(i + 1 < spw)
            def _():
                in_copy(i + 1).start()

            in_copy(i).wait()

            @pl.when(i >= 2)
            def _():
                out_copy(i - 2).wait()

            p = i % 2
            w127 = [
                wslab[p, F - 1, pl.ds(c * NLANES, NLANES)] for c in range(NCHUNK)
            ]
            for (lo, hi, km) in groups:
                cclip = float(K - km)

                def o_body(o, carry2, km=km, cclip=cclip):
                    accs = [w * cclip for w in w127] if km < K else [
                        wslab[p, o, pl.ds(c * NLANES, NLANES)]
                        for c in range(NCHUNK)
                    ]
                    for k in range(1 if km == K else 1, km + 1):
                        if km == K and k == 1:
                            continue
                        accs = [
                            a + wslab[p, k * o, pl.ds(c * NLANES, NLANES)]
                            for c, a in enumerate(accs)
                        ]
                    for c in range(NCHUNK):
                        oslab[p, o, pl.ds(c * NLANES, NLANES)] = accs[c]
                    return carry2

                lax.fori_loop(lo, hi, o_body, 0)
            out_copy(i).start()
            return carry

        lax.fori_loop(0, spw, slab_body, 0)
        out_copy(spw - 2).wait()
        out_copy(spw - 1).wait()

    return mix(win)


def kernel(x_in):
    win = _box_filter(x_in)
    return _harmonic_mix_sc(win)


# speedup vs baseline: 2.1357x; 1.0553x over previous
import functools

import jax
import jax.numpy as jnp
import numpy as np
from jax import lax
from jax.experimental import pallas as pl
from jax.experimental.pallas import tpu as pltpu
from jax.experimental.pallas import tpu_sc as plsc

K = 16
T = 8
TB = 32


FB = 32


def _box_kernel(x_ref, win_ref):
    x = x_ref[0]
    Tt = x.shape[0]
    z = jnp.zeros((T,) + x.shape[1:], x.dtype)
    xp = jnp.concatenate([z, x, z], axis=0)
    s2 = xp[:-1] + xp[1:]
    s4 = s2[:-2] + s2[2:]
    s8 = s4[:-4] + s4[4:]
    s16 = s8[:-8] + s8[8:]
    win = s16[:Tt] + xp[2 * T :]
    C = win.shape[-1]
    r = win.astype(jnp.bfloat16).astype(jnp.float32)
    bits = jax.lax.bitcast_convert_type(r, jnp.int32)
    lo = jax.lax.shift_right_logical(bits[:, :, : C // 2], 16)
    hi = bits[:, :, C // 2 :] & jnp.int32(-65536)
    win_ref[...] = (hi | lo)[None]


def _box_filter(x_in):
    B, Tt, F, C = x_in.shape
    return pl.pallas_call(
        _box_kernel,
        grid=(B, F // FB),
        in_specs=[pl.BlockSpec((1, Tt, FB, C), lambda b, f: (b, 0, f, 0))],
        out_specs=pl.BlockSpec((1, Tt, FB, C // 2), lambda b, f: (b, 0, f, 0)),
        out_shape=jax.ShapeDtypeStruct((B, Tt, F, C // 2), jnp.int32),
    )(x_in)


def _kmax_groups(F):
    groups = []
    for o in range(F):
        km = K if o == 0 else min(K, (F - 1) // o)
        if groups and groups[-1][2] == km:
            groups[-1] = (groups[-1][0], o + 1, km)
        else:
            groups.append((o, o + 1, km))
    return groups


def _harmonic_mix_sc(win, C):
    B, Tt, F, CP = win.shape
    NLANES = 16
    NCHP = CP // NLANES
    info = plsc.get_sparse_core_info()
    nworkers = info.num_cores * info.num_subcores
    nslabs = B * Tt
    spw = nslabs // nworkers
    groups = _kmax_groups(F)
    mesh = plsc.VectorSubcoreMesh(core_axis_name="c", subcore_axis_name="s")

    @functools.partial(
        pl.kernel,
        mesh=mesh,
        out_type=jax.ShapeDtypeStruct((B, Tt, F, C), jnp.float32),
        compiler_params=pltpu.CompilerParams(needs_layout_passes=False),
        scratch_types=[
            pltpu.VMEM((2, F, CP), jnp.int32),
            pltpu.VMEM((2, F, C), jnp.float32),
            pltpu.SemaphoreType.DMA((2,)),
            pltpu.SemaphoreType.DMA((2,)),
        ],
    )
    def mix(win_hbm, out_hbm, wslab, oslab, sem_in, sem_out):
        wid = lax.axis_index("s") * info.num_cores + lax.axis_index("c")
        base = wid * spw

        def load_row(p, f):
            out = []
            for ch in range(NCHP):
                v = wslab[p, f, pl.ds(ch * NLANES, NLANES)]
                out.append(
                    (
                        plsc.bitcast(v << 16, jnp.float32),
                        plsc.bitcast(v & jnp.int32(-65536), jnp.float32),
                    )
                )
            return out

        def in_copy(i):
            s = base + i
            p = i % 2
            return pltpu.make_async_copy(
                win_hbm.at[s // Tt, s % Tt], wslab.at[p], sem_in.at[p]
            )

        def out_copy(i):
            s = base + i
            p = i % 2
            return pltpu.make_async_copy(
                oslab.at[p], out_hbm.at[s // Tt, s % Tt], sem_out.at[p]
            )

        in_copy(0).start()

        def slab_body(i, carry):
            @pl.when(i + 1 < spw)
            def _():
                in_copy(i + 1).start()

            in_copy(i).wait()

            @pl.when(i >= 2)
            def _():
                out_copy(i - 2).wait()

            p = i % 2
            w127 = load_row(p, F - 1)
            for (lo, hi, km) in groups:
                cclip = float(K - km)

                def o_body(o, carry2, km=km, cclip=cclip):
                    if km < K:
                        accs = [(wl * cclip, wh * cclip) for (wl, wh) in w127]
                        k0 = 1
                    else:
                        accs = load_row(p, o)
                        k0 = 2
                    for k in range(k0, km + 1):
                        row = load_row(p, k * o)
                        accs = [
                            (al + rl, ah + rh)
                            for (al, ah), (rl, rh) in zip(accs, row)
                        ]
                    for ch in range(NCHP):
                        oslab[p, o, pl.ds(ch * NLANES, NLANES)] = accs[ch][0]
                        oslab[p, o, pl.ds(CP + ch * NLANES, NLANES)] = accs[ch][1]
                    return carry2

                lax.fori_loop(lo, hi, o_body, 0)
            out_copy(i).start()
            return carry

        lax.fori_loop(0, spw, slab_body, 0)
        out_copy(spw - 2).wait()
        out_copy(spw - 1).wait()

    return mix(win)


def kernel(x_in):
    win = _box_filter(x_in)
    return _harmonic_mix_sc(win, x_in.shape[-1])


# speedup vs baseline: 2.2413x; 1.0494x over previous
import functools

import jax
import jax.numpy as jnp
from jax import lax
from jax.experimental import pallas as pl
from jax.experimental.pallas import tpu as pltpu
from jax.experimental.pallas import tpu_sc as plsc

K = 16
T = 8
FB = 32


def _box_kernel(x_ref, win_ref):
    x = x_ref[0]
    Tt = x.shape[0]
    z = jnp.zeros((T,) + x.shape[1:], x.dtype)
    xp = jnp.concatenate([z, x, z], axis=0)
    s2 = xp[:-1] + xp[1:]
    s4 = s2[:-2] + s2[2:]
    s8 = s4[:-4] + s4[4:]
    s16 = s8[:-8] + s8[8:]
    win = s16[:Tt] + xp[2 * T :]
    r = win.astype(jnp.bfloat16).astype(jnp.float32)
    bits = jax.lax.bitcast_convert_type(r, jnp.int32)
    v2 = bits.reshape(Tt // 2, 2, *bits.shape[1:])
    packed = (v2[:, 1] & jnp.int32(-65536)) | jax.lax.shift_right_logical(
        v2[:, 0], 16
    )
    win_ref[...] = packed[None]


def _box_filter(x_in):
    B, Tt, F, C = x_in.shape
    return pl.pallas_call(
        _box_kernel,
        grid=(B, F // FB),
        in_specs=[pl.BlockSpec((1, Tt, FB, C), lambda b, f: (b, 0, f, 0))],
        out_specs=pl.BlockSpec((1, Tt // 2, FB, C), lambda b, f: (b, 0, f, 0)),
        out_shape=jax.ShapeDtypeStruct((B, Tt // 2, F, C), jnp.int32),
    )(x_in)


def _kmax_groups(F):
    groups = []
    for o in range(F):
        km = K if o == 0 else min(K, (F - 1) // o)
        if groups and groups[-1][2] == km:
            groups[-1] = (groups[-1][0], o + 1, km)
        else:
            groups.append((o, o + 1, km))
    return groups


def _harmonic_mix_sc(win, Tt):
    B, TP, F, C = win.shape
    NLANES = 16
    NCH = C // NLANES
    info = plsc.get_sparse_core_info()
    nworkers = info.num_cores * info.num_subcores
    npairs = B * TP
    ppw = npairs // nworkers
    groups = _kmax_groups(F)
    mesh = plsc.VectorSubcoreMesh(core_axis_name="c", subcore_axis_name="s")

    @functools.partial(
        pl.kernel,
        mesh=mesh,
        out_type=jax.ShapeDtypeStruct((B, Tt, F, C), jnp.float32),
        compiler_params=pltpu.CompilerParams(needs_layout_passes=False),
        scratch_types=[
            pltpu.VMEM((2, F, C), jnp.int32),
            pltpu.VMEM((2, 2, F, C), jnp.float32),
            pltpu.SemaphoreType.DMA((2,)),
            pltpu.SemaphoreType.DMA((2,)),
        ],
    )
    def mix(win_hbm, out_hbm, wslab, oslab, sem_in, sem_out):
        wid = lax.axis_index("s") * info.num_cores + lax.axis_index("c")
        base = wid * ppw

        def load_row(p, f):
            out = []
            for ch in range(NCH):
                v = wslab[p, f, pl.ds(ch * NLANES, NLANES)]
                out.append(
                    (
                        plsc.bitcast(v << 16, jnp.float32),
                        plsc.bitcast(v & jnp.int32(-65536), jnp.float32),
                    )
                )
            return out

        def in_copy(i):
            s = base + i
            p = i % 2
            return pltpu.make_async_copy(
                win_hbm.at[s // TP, s % TP], wslab.at[p], sem_in.at[p]
            )

        def out_copy(i):
            s = base + i
            p = i % 2
            return pltpu.make_async_copy(
                oslab.at[p], out_hbm.at[s // TP, pl.ds(2 * (s % TP), 2)],
                sem_out.at[p],
            )

        in_copy(0).start()

        def pair_body(i, carry):
            @pl.when(i + 1 < ppw)
            def _():
                in_copy(i + 1).start()

            in_copy(i).wait()

            @pl.when(i >= 2)
            def _():
                out_copy(i - 2).wait()

            p = i % 2
            w127 = load_row(p, F - 1)
            for (lo, hi, km) in groups:
                cclip = float(K - km)

                def o_body(o, carry2, km=km, cclip=cclip):
                    if km < K:
                        accs = [(wl * cclip, wh * cclip) for (wl, wh) in w127]
                        k0 = 1
                    else:
                        accs = load_row(p, o)
                        k0 = 2
                    for k in range(k0, km + 1):
                        row = load_row(p, k * o)
                        accs = [
                            (al + rl, ah + rh)
                            for (al, ah), (rl, rh) in zip(accs, row)
                        ]
                    for ch in range(NCH):
                        oslab[p, 0, o, pl.ds(ch * NLANES, NLANES)] = accs[ch][0]
                        oslab[p, 1, o, pl.ds(ch * NLANES, NLANES)] = accs[ch][1]
                    return carry2

                lax.fori_loop(lo, hi, o_body, 0)
            out_copy(i).start()
            return carry

        lax.fori_loop(0, ppw, pair_body, 0)
        out_copy(ppw - 2).wait()
        out_copy(ppw - 1).wait()

    return mix(win)


def kernel(x_in):
    win = _box_filter(x_in)
    return _harmonic_mix_sc(win, x_in.shape[1])
